# baseline (speedup 1.0000x reference)
import functools

import jax
import jax.numpy as jnp
from jax.experimental import pallas as pl
from jax.experimental.pallas import tpu as pltpu


def _moe_body(x_ref, xb_ref, gw_ref, gb_ref, w1_ref, b1_ref, w2_ref,
              b2_ref, gamma_ref, beta_ref, out_ref, c_s, acc_s, *, ne, nhc):
    e = pl.program_id(0)
    hc = pl.program_id(1)
    T = x_ref.shape[0]
    E = gw_ref.shape[1]

    @pl.when((e == 0) & (hc == 0))
    def _gate():
        logits = jnp.dot(x_ref[...], gw_ref[...],
                         preferred_element_type=jnp.float32) + gb_ref[...]
        ids = jax.lax.broadcasted_iota(jnp.int32, (T, E), 1)
        m1 = jnp.max(logits, axis=1, keepdims=True)
        i1 = jnp.min(jnp.where(logits == m1, ids, E), axis=1, keepdims=True)
        masked = jnp.where(ids == i1, -jnp.inf, logits)
        m2 = jnp.max(masked, axis=1, keepdims=True)
        i2 = jnp.min(jnp.where(masked == m2, ids, E), axis=1, keepdims=True)
        p1 = 1.0 / (1.0 + jnp.exp(m2 - m1))
        c_s[...] = jnp.where(ids == i1, p1, 0.0) + \
            jnp.where(ids == i2, 1.0 - p1, 0.0)
        acc_s[...] = jnp.zeros_like(acc_s)

    ids = jax.lax.broadcasted_iota(jnp.int32, (T, E), 1)
    ce = jnp.sum(jnp.where(ids == e, c_s[...], 0.0), axis=1, keepdims=True)

    h = jnp.dot(xb_ref[...], w1_ref[0],
                preferred_element_type=jnp.float32) + b1_ref[0]
    h = 0.5 * h * (1.0 + jnp.tanh(0.7978845608028654 * (h + 0.044715 * h * h * h)))
    hb = (h * ce).astype(jnp.bfloat16)
    acc_s[...] += jnp.dot(hb, w2_ref[0], preferred_element_type=jnp.float32)

    @pl.when(hc == 0)
    def _bias2():
        acc_s[...] += b2_ref[0] * ce

    @pl.when((e == ne - 1) & (hc == nhc - 1))
    def _finish():
        y = x_ref[...] + acc_s[...]
        mu = jnp.mean(y, axis=1, keepdims=True)
        var = jnp.mean((y - mu) ** 2, axis=1, keepdims=True)
        out_ref[...] = (y - mu) / jnp.sqrt(var + 1e-5) * gamma_ref[...] \
            + beta_ref[...]


def kernel(x, gate_w, gate_b, w1, b1, w2, b2, gamma, beta):
    B, T, D = x.shape
    E = gate_w.shape[1]
    H = w1.shape[2]
    HC = min(512, H)
    nhc = H // HC

    x2 = x.reshape(T, D)
    xb = x2.astype(jnp.bfloat16)
    w1b = w1.astype(jnp.bfloat16)
    w2b = w2.astype(jnp.bfloat16)
    b1r = b1.reshape(E, 1, H)
    b2r = b2.reshape(E, 1, D)
    gbr = gate_b.reshape(1, E)
    gammar = gamma.reshape(1, D)
    betar = beta.reshape(1, D)

    out = pl.pallas_call(
        functools.partial(_moe_body, ne=E, nhc=nhc),
        grid=(E, nhc),
        in_specs=[
            pl.BlockSpec((T, D), lambda e, hc: (0, 0)),
            pl.BlockSpec((T, D), lambda e, hc: (0, 0)),
            pl.BlockSpec((D, E), lambda e, hc: (0, 0)),
            pl.BlockSpec((1, E), lambda e, hc: (0, 0)),
            pl.BlockSpec((1, D, HC), lambda e, hc: (e, 0, hc)),
            pl.BlockSpec((1, 1, HC), lambda e, hc: (e, 0, hc)),
            pl.BlockSpec((1, HC, D), lambda e, hc: (e, hc, 0)),
            pl.BlockSpec((1, 1, D), lambda e, hc: (e, 0, 0)),
            pl.BlockSpec((1, D), lambda e, hc: (0, 0)),
            pl.BlockSpec((1, D), lambda e, hc: (0, 0)),
        ],
        out_specs=pl.BlockSpec((T, D), lambda e, hc: (0, 0)),
        out_shape=jax.ShapeDtypeStruct((T, D), jnp.float32),
        scratch_shapes=[
            pltpu.VMEM((T, E), jnp.float32),
            pltpu.VMEM((T, D), jnp.float32),
        ],
    )(x2, xb, gate_w, gbr, w1b, b1r, w2b, b2r, gammar, betar)
    return out.reshape(B, T, D)


# speedup vs baseline: 1.0743x; 1.0743x over previous
import functools

import jax
import jax.numpy as jnp
from jax.experimental import pallas as pl
from jax.experimental.pallas import tpu as pltpu


def _moe_body(x_ref, xb_ref, gw_ref, gb_ref, w1_ref, b1_ref, w2_ref,
              b2_ref, gamma_ref, beta_ref, out_ref, c_s, acc_s, *, ne, nhc):
    e = pl.program_id(0)
    hc = pl.program_id(1)
    T = x_ref.shape[0]
    E = gw_ref.shape[1]

    @pl.when((e == 0) & (hc == 0))
    def _gate():
        logits = jnp.dot(x_ref[...], gw_ref[...],
                         preferred_element_type=jnp.float32) + gb_ref[...]
        ids = jax.lax.broadcasted_iota(jnp.int32, (T, E), 1)
        m1 = jnp.max(logits, axis=1, keepdims=True)
        i1 = jnp.min(jnp.where(logits == m1, ids, E), axis=1, keepdims=True)
        masked = jnp.where(ids == i1, -jnp.inf, logits)
        m2 = jnp.max(masked, axis=1, keepdims=True)
        i2 = jnp.min(jnp.where(masked == m2, ids, E), axis=1, keepdims=True)
        p1 = 1.0 / (1.0 + jnp.exp(m2 - m1))
        c_s[...] = jnp.where(ids == i1, p1, 0.0) + \
            jnp.where(ids == i2, 1.0 - p1, 0.0)
        acc_s[...] = jnp.zeros_like(acc_s)

    ids = jax.lax.broadcasted_iota(jnp.int32, (T, E), 1)
    ce = jnp.sum(jnp.where(ids == e, c_s[...], 0.0), axis=1, keepdims=True)

    h = jnp.dot(xb_ref[...], w1_ref[0],
                preferred_element_type=jnp.float32) + b1_ref[0]
    h = 0.5 * h * (1.0 + jax.lax.erf(h * 0.7071067811865476))
    hb = (h * ce).astype(jnp.bfloat16)
    acc_s[...] += jnp.dot(hb, w2_ref[0], preferred_element_type=jnp.float32)

    @pl.when(hc == 0)
    def _bias2():
        acc_s[...] += b2_ref[0] * ce

    @pl.when((e == ne - 1) & (hc == nhc - 1))
    def _finish():
        y = x_ref[...] + acc_s[...]
        mu = jnp.mean(y, axis=1, keepdims=True)
        var = jnp.mean((y - mu) ** 2, axis=1, keepdims=True)
        out_ref[...] = (y - mu) / jnp.sqrt(var + 1e-5) * gamma_ref[...] \
            + beta_ref[...]


def kernel(x, gate_w, gate_b, w1, b1, w2, b2, gamma, beta):
    B, T, D = x.shape
    E = gate_w.shape[1]
    H = w1.shape[2]
    HC = min(1024, H)
    nhc = H // HC

    x2 = x.reshape(T, D)
    xb = x2.astype(jnp.bfloat16)
    w1b = w1.astype(jnp.bfloat16)
    w2b = w2.astype(jnp.bfloat16)
    b1r = b1.reshape(E, 1, H)
    b2r = b2.reshape(E, 1, D)
    gbr = gate_b.reshape(1, E)
    gammar = gamma.reshape(1, D)
    betar = beta.reshape(1, D)

    out = pl.pallas_call(
        functools.partial(_moe_body, ne=E, nhc=nhc),
        grid=(E, nhc),
        in_specs=[
            pl.BlockSpec((T, D), lambda e, hc: (0, 0)),
            pl.BlockSpec((T, D), lambda e, hc: (0, 0)),
            pl.BlockSpec((D, E), lambda e, hc: (0, 0)),
            pl.BlockSpec((1, E), lambda e, hc: (0, 0)),
            pl.BlockSpec((1, D, HC), lambda e, hc: (e, 0, hc)),
            pl.BlockSpec((1, 1, HC), lambda e, hc: (e, 0, hc)),
            pl.BlockSpec((1, HC, D), lambda e, hc: (e, hc, 0)),
            pl.BlockSpec((1, 1, D), lambda e, hc: (e, 0, 0)),
            pl.BlockSpec((1, D), lambda e, hc: (0, 0)),
            pl.BlockSpec((1, D), lambda e, hc: (0, 0)),
        ],
        out_specs=pl.BlockSpec((T, D), lambda e, hc: (0, 0)),
        out_shape=jax.ShapeDtypeStruct((T, D), jnp.float32),
        scratch_shapes=[
            pltpu.VMEM((T, E), jnp.float32),
            pltpu.VMEM((T, D), jnp.float32),
        ],
    )(x2, xb, gate_w, gbr, w1b, b1r, w2b, b2r, gammar, betar)
    return out.reshape(B, T, D)


# speedup vs baseline: 1.5298x; 1.4241x over previous
import functools

import jax
import jax.numpy as jnp
from jax import lax
from jax.experimental import pallas as pl
from jax.experimental.pallas import tpu as pltpu
from jax.experimental.pallas import tpu_sc as plsc

BLK = 256
NSUB = 16
LANES = 16


def _gate_body(x_ref, gw_ref, gb_ref, e0_ref, e1_ref, w0_ref, w1_ref):
    T = x_ref.shape[0]
    E = gw_ref.shape[1]
    xv = x_ref[...]
    logits = jnp.dot(xv, gw_ref[...],
                     preferred_element_type=jnp.float32) + gb_ref[...]
    ids = lax.broadcasted_iota(jnp.int32, (T, E), 1)
    m1 = jnp.max(logits, axis=1, keepdims=True)
    i1 = jnp.min(jnp.where(logits == m1, ids, E), axis=1, keepdims=True)
    masked = jnp.where(ids == i1, -jnp.inf, logits)
    m2 = jnp.max(masked, axis=1, keepdims=True)
    i2 = jnp.min(jnp.where(masked == m2, ids, E), axis=1, keepdims=True)
    p1 = 1.0 / (1.0 + jnp.exp(m2 - m1))
    e0_ref[...] = i1
    e1_ref[...] = i2
    w0_ref[...] = p1
    w1_ref[...] = 1.0 - p1


def _hist_body(e0_hbm, e1_hbm, hist_hbm, eb0, eb1, histv, sem, *, tpw,
               nexp):
    wid = lax.axis_index("s")
    nvec = tpw // LANES
    rb = wid * nvec
    lane = lax.broadcasted_iota(jnp.int32, (LANES,), 0)
    pltpu.sync_copy(e0_hbm.at[pl.ds(rb, nvec)], eb0)
    pltpu.sync_copy(e1_hbm.at[pl.ds(rb, nvec)], eb1)
    hist = jnp.zeros((LANES,), jnp.int32)
    for ref in (eb0, eb1):
        for v in range(nvec):
            vec = ref[v]
            for e in range(nexp):
                cnt = plsc.all_reduce_population_count(vec == e)
                hist = hist + jnp.where(lane == e, cnt, 0)
    histv[...] = hist
    pltpu.sync_copy(histv, hist_hbm.at[wid])


def _route_body(e0_hbm, e1_hbm, hist_hbm, xb_hbm, pos0_hbm, pos1_hbm,
                xs_hbm, be_hbm, na_hbm, eb0, eb1, histall, posb0, posb1,
                bebuf, nabuf, xbuf, sem, *, tpw, nexp):
    wid = lax.axis_index("s")
    rb = wid * (tpw // LANES)
    nvec = tpw // LANES
    lane = lax.broadcasted_iota(jnp.int32, (LANES,), 0)

    pltpu.sync_copy(e0_hbm.at[pl.ds(rb, nvec)], eb0)
    pltpu.sync_copy(e1_hbm.at[pl.ds(rb, nvec)], eb1)
    pltpu.sync_copy(hist_hbm, histall)

    tot = jnp.zeros((LANES,), jnp.int32)
    pref = jnp.zeros((LANES,), jnp.int32)
    for w in range(NSUB):
        row = histall[w]
        tot = tot + row
        pref = pref + row * jnp.where(w < wid, 1, 0)

    padded = ((tot + (BLK - 1)) >> 8) << 8
    start = plsc.cumsum(padded) - padded
    base = start + pref
    nblk = padded >> 8
    gstart = plsc.cumsum(nblk) - nblk
    na_scal = jnp.sum(nblk)

    bg0 = jnp.zeros((LANES,), jnp.int32)
    bg1 = jnp.zeros((LANES,), jnp.int32)
    gv0 = lane
    gv1 = lane + LANES
    for e in range(nexp):
        ge = jnp.sum(jnp.where(lane == e, gstart, 0))
        bg0 = bg0 + (gv0 >= ge).astype(jnp.int32)
        bg1 = bg1 + (gv1 >= ge).astype(jnp.int32)

    @pl.when(wid == 0)
    def _write_meta():
        bebuf[0] = bg0 - 1
        bebuf[1] = bg1 - 1
        nabuf[0] = jnp.zeros((LANES,), jnp.int32) + na_scal
        pltpu.sync_copy(bebuf, be_hbm)
        pltpu.sync_copy(nabuf, na_hbm)

    counter = jnp.zeros((LANES,), jnp.int32)
    for ref, pbuf in ((eb0, posb0), (eb1, posb1)):
        for v in range(nvec):
            vec = ref[v]
            posv = jnp.zeros((LANES,), jnp.int32)
            for e in range(nexp):
                m = vec == e
                rank = plsc.cumsum(m.astype(jnp.int32)) - 1
                bc = jnp.sum(jnp.where(lane == e, base + counter, 0))
                posv = jnp.where(m, bc + rank, posv)
                counter = counter + jnp.where(
                    lane == e, plsc.all_reduce_population_count(m), 0)
            pbuf[v] = posv
    pltpu.sync_copy(posb0, pos0_hbm.at[pl.ds(rb, nvec)])
    pltpu.sync_copy(posb1, pos1_hbm.at[pl.ds(rb, nvec)])

    for c in range(nvec):
        tb = wid * tpw + c * LANES
        pltpu.sync_copy(xb_hbm.at[pl.ds(tb, LANES)], xbuf)
        pltpu.async_copy(xbuf, xs_hbm.at[posb0.at[c]], sem).wait()
        pltpu.async_copy(xbuf, xs_hbm.at[posb1.at[c]], sem).wait()


def _ffn_body(be_ref, na_ref, xs_ref, w1_ref, b1_ref, w2_ref, b2_ref,
              ys_ref):
    g = pl.program_id(0)

    @pl.when(g < na_ref[0])
    def _():
        h = jnp.dot(xs_ref[...].astype(jnp.bfloat16), w1_ref[0],
                    preferred_element_type=jnp.float32) + b1_ref[0]
        h = 0.5 * h * (1.0 + lax.erf(h * 0.7071067811865476))
        ys_ref[...] = jnp.dot(h.astype(jnp.bfloat16), w2_ref[0],
                              preferred_element_type=jnp.float32) \
            + b2_ref[0]


def _gather_body(ys_hbm, pos0_hbm, pos1_hbm, ys0_hbm, ys1_hbm, posb0,
                 posb1, ybuf, sem, *, tpw):
    wid = lax.axis_index("s")
    nvec = tpw // LANES
    rb = wid * nvec
    pltpu.sync_copy(pos0_hbm.at[pl.ds(rb, nvec)], posb0)
    pltpu.sync_copy(pos1_hbm.at[pl.ds(rb, nvec)], posb1)
    for c in range(nvec):
        tb = wid * tpw + c * LANES
        pltpu.async_copy(ys_hbm.at[posb0.at[c]], ybuf, sem).wait()
        pltpu.sync_copy(ybuf, ys0_hbm.at[pl.ds(tb, LANES)])
        pltpu.async_copy(ys_hbm.at[posb1.at[c]], ybuf, sem).wait()
        pltpu.sync_copy(ybuf, ys1_hbm.at[pl.ds(tb, LANES)])


def _out_body(x_ref, y0_ref, y1_ref, w0_ref, w1_ref, gamma_ref, beta_ref,
              o_ref):
    y = x_ref[...] + w0_ref[...] * y0_ref[...] + w1_ref[...] * y1_ref[...]
    mu = jnp.mean(y, axis=1, keepdims=True)
    var = jnp.mean((y - mu) ** 2, axis=1, keepdims=True)
    o_ref[...] = (y - mu) / jnp.sqrt(var + 1e-5) * gamma_ref[...] \
        + beta_ref[...]


def kernel(x, gate_w, gate_b, w1, b1, w2, b2, gamma, beta):
    B, T, D = x.shape
    E = gate_w.shape[1]
    H = w1.shape[2]
    gmax = (2 * T) // BLK + E - 1
    nrows = gmax * BLK
    tpw = T // NSUB

    x2 = x.reshape(T, D)
    w1b = w1.astype(jnp.bfloat16)
    w2b = w2.astype(jnp.bfloat16)
    b1r = b1.reshape(E, 1, H)
    b2r = b2.reshape(E, 1, D)
    gbr = gate_b.reshape(1, E)
    gammar = gamma.reshape(1, D)
    betar = beta.reshape(1, D)

    e0, e1, w0c, w1c = pl.pallas_call(
        _gate_body,
        in_specs=[pl.BlockSpec((T, D), lambda: (0, 0)),
                  pl.BlockSpec((D, E), lambda: (0, 0)),
                  pl.BlockSpec((1, E), lambda: (0, 0))],
        out_specs=[pl.BlockSpec((T, 1), lambda: (0, 0)),
                   pl.BlockSpec((T, 1), lambda: (0, 0)),
                   pl.BlockSpec((T, 1), lambda: (0, 0)),
                   pl.BlockSpec((T, 1), lambda: (0, 0))],
        out_shape=[jax.ShapeDtypeStruct((T, 1), jnp.int32),
                   jax.ShapeDtypeStruct((T, 1), jnp.int32),
                   jax.ShapeDtypeStruct((T, 1), jnp.float32),
                   jax.ShapeDtypeStruct((T, 1), jnp.float32)],
    )(x2, gate_w, gbr)

    e0r = e0.reshape(T // LANES, LANES)
    e1r = e1.reshape(T // LANES, LANES)

    mesh = plsc.VectorSubcoreMesh(core_axis_name="c", subcore_axis_name="s",
                                  num_cores=1)
    histk = pl.kernel(
        functools.partial(_hist_body, tpw=tpw, nexp=E),
        mesh=mesh,
        compiler_params=pltpu.CompilerParams(needs_layout_passes=False),
        out_type=jax.ShapeDtypeStruct((NSUB, LANES), jnp.int32),
        scratch_types=[
            pltpu.VMEM((tpw // LANES, LANES), jnp.int32),
            pltpu.VMEM((tpw // LANES, LANES), jnp.int32),
            pltpu.VMEM((LANES,), jnp.int32),
            pltpu.SemaphoreType.DMA,
        ],
    )
    histtab = histk(e0r, e1r)
    route = pl.kernel(
        functools.partial(_route_body, tpw=tpw, nexp=E),
        mesh=mesh,
        compiler_params=pltpu.CompilerParams(needs_layout_passes=False),
        out_type=(jax.ShapeDtypeStruct((T // LANES, LANES), jnp.int32),
                  jax.ShapeDtypeStruct((T // LANES, LANES), jnp.int32),
                  jax.ShapeDtypeStruct((nrows, D), jnp.float32),
                  jax.ShapeDtypeStruct((2, LANES), jnp.int32),
                  jax.ShapeDtypeStruct((1, LANES), jnp.int32)),
        scratch_types=[
            pltpu.VMEM((tpw // LANES, LANES), jnp.int32),
            pltpu.VMEM((tpw // LANES, LANES), jnp.int32),
            pltpu.VMEM((NSUB, LANES), jnp.int32),
            pltpu.VMEM((tpw // LANES, LANES), jnp.int32),
            pltpu.VMEM((tpw // LANES, LANES), jnp.int32),
            pltpu.VMEM((2, LANES), jnp.int32),
            pltpu.VMEM((1, LANES), jnp.int32),
            pltpu.VMEM((LANES, D), jnp.float32),
            pltpu.SemaphoreType.DMA,
        ],
    )
    pos0, pos1, xs, be2, na2 = route(e0r, e1r, histtab, x2)

    ys = pl.pallas_call(
        _ffn_body,
        grid_spec=pltpu.PrefetchScalarGridSpec(
            num_scalar_prefetch=2,
            grid=(gmax,),
            in_specs=[
                pl.BlockSpec((BLK, D), lambda g, be, na: (g, 0)),
                pl.BlockSpec((1, D, H), lambda g, be, na: (be[g], 0, 0)),
                pl.BlockSpec((1, 1, H), lambda g, be, na: (be[g], 0, 0)),
                pl.BlockSpec((1, H, D), lambda g, be, na: (be[g], 0, 0)),
                pl.BlockSpec((1, 1, D), lambda g, be, na: (be[g], 0, 0)),
            ],
            out_specs=pl.BlockSpec((BLK, D), lambda g, be, na: (g, 0)),
        ),
        out_shape=jax.ShapeDtypeStruct((nrows, D), jnp.float32),
    )(be2.reshape(2 * LANES), na2.reshape(LANES), xs, w1b, b1r, w2b, b2r)

    gather = pl.kernel(
        functools.partial(_gather_body, tpw=tpw),
        mesh=plsc.VectorSubcoreMesh(core_axis_name="c",
                                    subcore_axis_name="s", num_cores=1),
        compiler_params=pltpu.CompilerParams(needs_layout_passes=False),
        out_type=(jax.ShapeDtypeStruct((T, D), jnp.float32),
                  jax.ShapeDtypeStruct((T, D), jnp.float32)),
        scratch_types=[
            pltpu.VMEM((tpw // LANES, LANES), jnp.int32),
            pltpu.VMEM((tpw // LANES, LANES), jnp.int32),
            pltpu.VMEM((LANES, D), jnp.float32),
            pltpu.SemaphoreType.DMA,
        ],
    )
    ys0, ys1 = gather(ys, pos0, pos1)

    out = pl.pallas_call(
        _out_body,
        in_specs=[pl.BlockSpec((T, D), lambda: (0, 0)),
                  pl.BlockSpec((T, D), lambda: (0, 0)),
                  pl.BlockSpec((T, D), lambda: (0, 0)),
                  pl.BlockSpec((T, 1), lambda: (0, 0)),
                  pl.BlockSpec((T, 1), lambda: (0, 0)),
                  pl.BlockSpec((1, D), lambda: (0, 0)),
                  pl.BlockSpec((1, D), lambda: (0, 0))],
        out_specs=pl.BlockSpec((T, D), lambda: (0, 0)),
        out_shape=jax.ShapeDtypeStruct((T, D), jnp.float32),
    )(x2, ys0, ys1, w0c, w1c, gammar, betar)
    return out.reshape(B, T, D)


# speedup vs baseline: 1.5877x; 1.0378x over previous
import functools

import jax
import jax.numpy as jnp
from jax import lax
from jax.experimental import pallas as pl
from jax.experimental.pallas import tpu as pltpu
from jax.experimental.pallas import tpu_sc as plsc

BLK = 256
NWORK = 32
LANES = 16


def _gate_body(x_ref, gw_ref, gb_ref, e0_ref, e1_ref, w0_ref, w1_ref):
    T = x_ref.shape[0]
    E = gw_ref.shape[1]
    xv = x_ref[...]
    logits = jnp.dot(xv, gw_ref[...],
                     preferred_element_type=jnp.float32) + gb_ref[...]
    ids = lax.broadcasted_iota(jnp.int32, (T, E), 1)
    m1 = jnp.max(logits, axis=1, keepdims=True)
    i1 = jnp.min(jnp.where(logits == m1, ids, E), axis=1, keepdims=True)
    masked = jnp.where(ids == i1, -jnp.inf, logits)
    m2 = jnp.max(masked, axis=1, keepdims=True)
    i2 = jnp.min(jnp.where(masked == m2, ids, E), axis=1, keepdims=True)
    p1 = 1.0 / (1.0 + jnp.exp(m2 - m1))
    e0_ref[...] = i1
    e1_ref[...] = i2
    w0_ref[...] = p1
    w1_ref[...] = 1.0 - p1


def _hist_body(e0_hbm, e1_hbm, hist_hbm, eb0, eb1, histv, sem, *, tpw,
               nexp):
    wid = lax.axis_index("s") * 2 + lax.axis_index("c")
    nvec = tpw // LANES
    rb = wid * nvec
    lane = lax.broadcasted_iota(jnp.int32, (LANES,), 0)
    pltpu.sync_copy(e0_hbm.at[pl.ds(rb, nvec)], eb0)
    pltpu.sync_copy(e1_hbm.at[pl.ds(rb, nvec)], eb1)
    hist = jnp.zeros((LANES,), jnp.int32)
    for ref in (eb0, eb1):
        for v in range(nvec):
            vec = ref[v]
            for e in range(nexp):
                cnt = plsc.all_reduce_population_count(vec == e)
                hist = hist + jnp.where(lane == e, cnt, 0)
    histv[...] = hist
    pltpu.sync_copy(histv, hist_hbm.at[wid])


def _route_body(e0_hbm, e1_hbm, hist_hbm, xb_hbm, pos0_hbm, pos1_hbm,
                xs_hbm, be_hbm, na_hbm, eb0, eb1, histall, posb0, posb1,
                bebuf, nabuf, xbuf, sem, *, tpw, nexp):
    wid = lax.axis_index("s") * 2 + lax.axis_index("c")
    rb = wid * (tpw // LANES)
    nvec = tpw // LANES
    lane = lax.broadcasted_iota(jnp.int32, (LANES,), 0)

    pltpu.sync_copy(e0_hbm.at[pl.ds(rb, nvec)], eb0)
    pltpu.sync_copy(e1_hbm.at[pl.ds(rb, nvec)], eb1)
    pltpu.sync_copy(hist_hbm, histall)

    tot = jnp.zeros((LANES,), jnp.int32)
    pref = jnp.zeros((LANES,), jnp.int32)
    for w in range(NWORK):
        row = histall[w]
        tot = tot + row
        pref = pref + row * jnp.where(w < wid, 1, 0)

    padded = ((tot + (BLK - 1)) >> 8) << 8
    start = plsc.cumsum(padded) - padded
    base = start + pref
    nblk = padded >> 8
    gstart = plsc.cumsum(nblk) - nblk
    na_scal = jnp.sum(nblk)

    bg0 = jnp.zeros((LANES,), jnp.int32)
    bg1 = jnp.zeros((LANES,), jnp.int32)
    gv0 = lane
    gv1 = lane + LANES
    for e in range(nexp):
        ge = jnp.sum(jnp.where(lane == e, gstart, 0))
        bg0 = bg0 + (gv0 >= ge).astype(jnp.int32)
        bg1 = bg1 + (gv1 >= ge).astype(jnp.int32)

    @pl.when(wid == 0)
    def _write_meta():
        bebuf[0] = bg0 - 1
        bebuf[1] = bg1 - 1
        nabuf[0] = jnp.zeros((LANES,), jnp.int32) + na_scal
        pltpu.sync_copy(bebuf, be_hbm)
        pltpu.sync_copy(nabuf, na_hbm)

    counter = jnp.zeros((LANES,), jnp.int32)
    for ref, pbuf in ((eb0, posb0), (eb1, posb1)):
        for v in range(nvec):
            vec = ref[v]
            posv = jnp.zeros((LANES,), jnp.int32)
            for e in range(nexp):
                m = vec == e
                rank = plsc.cumsum(m.astype(jnp.int32)) - 1
                bc = jnp.sum(jnp.where(lane == e, base + counter, 0))
                posv = jnp.where(m, bc + rank, posv)
                counter = counter + jnp.where(
                    lane == e, plsc.all_reduce_population_count(m), 0)
            pbuf[v] = posv
    pltpu.sync_copy(posb0, pos0_hbm.at[pl.ds(rb, nvec)])
    pltpu.sync_copy(posb1, pos1_hbm.at[pl.ds(rb, nvec)])

    for c in range(nvec):
        tb = wid * tpw + c * LANES
        pltpu.sync_copy(xb_hbm.at[pl.ds(tb, LANES)], xbuf)
        pltpu.async_copy(xbuf, xs_hbm.at[posb0.at[c]], sem).wait()
        pltpu.async_copy(xbuf, xs_hbm.at[posb1.at[c]], sem).wait()


def _ffn_body(be_ref, na_ref, xs_ref, w1_ref, b1_ref, w2_ref, b2_ref,
              ys_ref):
    g = pl.program_id(0)

    @pl.when(g < na_ref[0])
    def _():
        h = jnp.dot(xs_ref[...].astype(jnp.bfloat16), w1_ref[0],
                    preferred_element_type=jnp.float32) + b1_ref[0]
        h = 0.5 * h * (1.0 + lax.erf(h * 0.7071067811865476))
        ys_ref[...] = jnp.dot(h.astype(jnp.bfloat16), w2_ref[0],
                              preferred_element_type=jnp.float32) \
            + b2_ref[0]


def _gather_body(ys_hbm, pos0_hbm, pos1_hbm, ys0_hbm, ys1_hbm, posb0,
                 posb1, ybuf, sem, *, tpw):
    wid = lax.axis_index("s") * 2 + lax.axis_index("c")
    nvec = tpw // LANES
    rb = wid * nvec
    pltpu.sync_copy(pos0_hbm.at[pl.ds(rb, nvec)], posb0)
    pltpu.sync_copy(pos1_hbm.at[pl.ds(rb, nvec)], posb1)
    for c in range(nvec):
        tb = wid * tpw + c * LANES
        pltpu.async_copy(ys_hbm.at[posb0.at[c]], ybuf, sem).wait()
        pltpu.sync_copy(ybuf, ys0_hbm.at[pl.ds(tb, LANES)])
        pltpu.async_copy(ys_hbm.at[posb1.at[c]], ybuf, sem).wait()
        pltpu.sync_copy(ybuf, ys1_hbm.at[pl.ds(tb, LANES)])


def _out_body(x_ref, y0_ref, y1_ref, w0_ref, w1_ref, gamma_ref, beta_ref,
              o_ref):
    y = x_ref[...] + w0_ref[...] * y0_ref[...] + w1_ref[...] * y1_ref[...]
    mu = jnp.mean(y, axis=1, keepdims=True)
    var = jnp.mean((y - mu) ** 2, axis=1, keepdims=True)
    o_ref[...] = (y - mu) / jnp.sqrt(var + 1e-5) * gamma_ref[...] \
        + beta_ref[...]


def kernel(x, gate_w, gate_b, w1, b1, w2, b2, gamma, beta):
    B, T, D = x.shape
    E = gate_w.shape[1]
    H = w1.shape[2]
    gmax = (2 * T) // BLK + E - 1
    nrows = gmax * BLK
    tpw = T // NWORK

    x2 = x.reshape(T, D)
    w1b = w1.astype(jnp.bfloat16)
    w2b = w2.astype(jnp.bfloat16)
    b1r = b1.reshape(E, 1, H)
    b2r = b2.reshape(E, 1, D)
    gbr = gate_b.reshape(1, E)
    gammar = gamma.reshape(1, D)
    betar = beta.reshape(1, D)

    e0, e1, w0c, w1c = pl.pallas_call(
        _gate_body,
        in_specs=[pl.BlockSpec((T, D), lambda: (0, 0)),
                  pl.BlockSpec((D, E), lambda: (0, 0)),
                  pl.BlockSpec((1, E), lambda: (0, 0))],
        out_specs=[pl.BlockSpec((T, 1), lambda: (0, 0)),
                   pl.BlockSpec((T, 1), lambda: (0, 0)),
                   pl.BlockSpec((T, 1), lambda: (0, 0)),
                   pl.BlockSpec((T, 1), lambda: (0, 0))],
        out_shape=[jax.ShapeDtypeStruct((T, 1), jnp.int32),
                   jax.ShapeDtypeStruct((T, 1), jnp.int32),
                   jax.ShapeDtypeStruct((T, 1), jnp.float32),
                   jax.ShapeDtypeStruct((T, 1), jnp.float32)],
    )(x2, gate_w, gbr)

    e0r = e0.reshape(T // LANES, LANES)
    e1r = e1.reshape(T // LANES, LANES)

    mesh = plsc.VectorSubcoreMesh(core_axis_name="c", subcore_axis_name="s",
                                  num_cores=2)
    histk = pl.kernel(
        functools.partial(_hist_body, tpw=tpw, nexp=E),
        mesh=mesh,
        compiler_params=pltpu.CompilerParams(needs_layout_passes=False),
        out_type=jax.ShapeDtypeStruct((NWORK, LANES), jnp.int32),
        scratch_types=[
            pltpu.VMEM((tpw // LANES, LANES), jnp.int32),
            pltpu.VMEM((tpw // LANES, LANES), jnp.int32),
            pltpu.VMEM((LANES,), jnp.int32),
            pltpu.SemaphoreType.DMA,
        ],
    )
    histtab = histk(e0r, e1r)
    route = pl.kernel(
        functools.partial(_route_body, tpw=tpw, nexp=E),
        mesh=mesh,
        compiler_params=pltpu.CompilerParams(needs_layout_passes=False),
        out_type=(jax.ShapeDtypeStruct((T // LANES, LANES), jnp.int32),
                  jax.ShapeDtypeStruct((T // LANES, LANES), jnp.int32),
                  jax.ShapeDtypeStruct((nrows, D), jnp.float32),
                  jax.ShapeDtypeStruct((2, LANES), jnp.int32),
                  jax.ShapeDtypeStruct((1, LANES), jnp.int32)),
        scratch_types=[
            pltpu.VMEM((tpw // LANES, LANES), jnp.int32),
            pltpu.VMEM((tpw // LANES, LANES), jnp.int32),
            pltpu.VMEM((NWORK, LANES), jnp.int32),
            pltpu.VMEM((tpw // LANES, LANES), jnp.int32),
            pltpu.VMEM((tpw // LANES, LANES), jnp.int32),
            pltpu.VMEM((2, LANES), jnp.int32),
            pltpu.VMEM((1, LANES), jnp.int32),
            pltpu.VMEM((LANES, D), jnp.float32),
            pltpu.SemaphoreType.DMA,
        ],
    )
    pos0, pos1, xs, be2, na2 = route(e0r, e1r, histtab, x2)

    ys = pl.pallas_call(
        _ffn_body,
        grid_spec=pltpu.PrefetchScalarGridSpec(
            num_scalar_prefetch=2,
            grid=(gmax,),
            in_specs=[
                pl.BlockSpec((BLK, D), lambda g, be, na: (g, 0)),
                pl.BlockSpec((1, D, H), lambda g, be, na: (be[g], 0, 0)),
                pl.BlockSpec((1, 1, H), lambda g, be, na: (be[g], 0, 0)),
                pl.BlockSpec((1, H, D), lambda g, be, na: (be[g], 0, 0)),
                pl.BlockSpec((1, 1, D), lambda g, be, na: (be[g], 0, 0)),
            ],
            out_specs=pl.BlockSpec((BLK, D), lambda g, be, na: (g, 0)),
        ),
        out_shape=jax.ShapeDtypeStruct((nrows, D), jnp.float32),
    )(be2.reshape(2 * LANES), na2.reshape(LANES), xs, w1b, b1r, w2b, b2r)

    gather = pl.kernel(
        functools.partial(_gather_body, tpw=tpw),
        mesh=plsc.VectorSubcoreMesh(core_axis_name="c",
                                    subcore_axis_name="s", num_cores=2),
        compiler_params=pltpu.CompilerParams(needs_layout_passes=False),
        out_type=(jax.ShapeDtypeStruct((T, D), jnp.float32),
                  jax.ShapeDtypeStruct((T, D), jnp.float32)),
        scratch_types=[
            pltpu.VMEM((tpw // LANES, LANES), jnp.int32),
            pltpu.VMEM((tpw // LANES, LANES), jnp.int32),
            pltpu.VMEM((LANES, D), jnp.float32),
            pltpu.SemaphoreType.DMA,
        ],
    )
    ys0, ys1 = gather(ys, pos0, pos1)

    out = pl.pallas_call(
        _out_body,
        in_specs=[pl.BlockSpec((T, D), lambda: (0, 0)),
                  pl.BlockSpec((T, D), lambda: (0, 0)),
                  pl.BlockSpec((T, D), lambda: (0, 0)),
                  pl.BlockSpec((T, 1), lambda: (0, 0)),
                  pl.BlockSpec((T, 1), lambda: (0, 0)),
                  pl.BlockSpec((1, D), lambda: (0, 0)),
                  pl.BlockSpec((1, D), lambda: (0, 0))],
        out_specs=pl.BlockSpec((T, D), lambda: (0, 0)),
        out_shape=jax.ShapeDtypeStruct((T, D), jnp.float32),
    )(x2, ys0, ys1, w0c, w1c, gammar, betar)
    return out.reshape(B, T, D)


# speedup vs baseline: 1.6183x; 1.0193x over previous
import functools

import jax
import jax.numpy as jnp
from jax import lax
from jax.experimental import pallas as pl
from jax.experimental.pallas import tpu as pltpu
from jax.experimental.pallas import tpu_sc as plsc

BLK = 256
NWORK = 32
LANES = 16


def _gate_body(x_ref, gw_ref, gb_ref, e0_ref, e1_ref, w0_ref, w1_ref,
               hist_ref, *, shift):
    T = x_ref.shape[0]
    E = gw_ref.shape[1]
    xv = x_ref[...]
    logits = jnp.dot(xv, gw_ref[...],
                     preferred_element_type=jnp.float32) + gb_ref[...]
    ids = lax.broadcasted_iota(jnp.int32, (T, E), 1)
    m1 = jnp.max(logits, axis=1, keepdims=True)
    i1 = jnp.min(jnp.where(logits == m1, ids, E), axis=1, keepdims=True)
    masked = jnp.where(ids == i1, -jnp.inf, logits)
    m2 = jnp.max(masked, axis=1, keepdims=True)
    i2 = jnp.min(jnp.where(masked == m2, ids, E), axis=1, keepdims=True)
    p1 = 1.0 / (1.0 + jnp.exp(m2 - m1))
    e0_ref[...] = i1
    e1_ref[...] = i2
    w0_ref[...] = p1
    w1_ref[...] = 1.0 - p1
    ids16 = lax.broadcasted_iota(jnp.int32, (T, LANES), 1)
    onehot = (ids16 == i1).astype(jnp.float32) \
        + (ids16 == i2).astype(jnp.float32)
    wrow = lax.broadcasted_iota(jnp.int32, (NWORK, T), 0)
    tcol = lax.broadcasted_iota(jnp.int32, (NWORK, T), 1)
    sel = ((tcol >> shift) == wrow).astype(jnp.float32)
    hist_ref[...] = jnp.dot(sel, onehot,
                            preferred_element_type=jnp.float32
                            ).astype(jnp.int32)


def _route_body(e0_hbm, e1_hbm, hist_hbm, xb_hbm, pos0_hbm, pos1_hbm,
                xs_hbm, be_hbm, na_hbm, eb0, eb1, histall, posb0, posb1,
                bebuf, nabuf, xbuf, sem, *, tpw, nexp):
    wid = lax.axis_index("s") * 2 + lax.axis_index("c")
    rb = wid * (tpw // LANES)
    nvec = tpw // LANES
    lane = lax.broadcasted_iota(jnp.int32, (LANES,), 0)

    pltpu.sync_copy(e0_hbm.at[pl.ds(rb, nvec)], eb0)
    pltpu.sync_copy(e1_hbm.at[pl.ds(rb, nvec)], eb1)
    pltpu.sync_copy(hist_hbm, histall)

    tot = jnp.zeros((LANES,), jnp.int32)
    pref = jnp.zeros((LANES,), jnp.int32)
    for w in range(NWORK):
        row = histall[w]
        tot = tot + row
        pref = pref + row * jnp.where(w < wid, 1, 0)

    padded = ((tot + (BLK - 1)) >> 8) << 8
    start = plsc.cumsum(padded) - padded
    base = start + pref
    nblk = padded >> 8
    gstart = plsc.cumsum(nblk) - nblk
    na_scal = jnp.sum(nblk)

    bg0 = jnp.zeros((LANES,), jnp.int32)
    bg1 = jnp.zeros((LANES,), jnp.int32)
    gv0 = lane
    gv1 = lane + LANES
    for e in range(nexp):
        ge = jnp.sum(jnp.where(lane == e, gstart, 0))
        bg0 = bg0 + (gv0 >= ge).astype(jnp.int32)
        bg1 = bg1 + (gv1 >= ge).astype(jnp.int32)

    @pl.when(wid == 0)
    def _write_meta():
        bebuf[0] = bg0 - 1
        bebuf[1] = bg1 - 1
        nabuf[0] = jnp.zeros((LANES,), jnp.int32) + na_scal
        pltpu.sync_copy(bebuf, be_hbm)
        pltpu.sync_copy(nabuf, na_hbm)

    counter = jnp.zeros((LANES,), jnp.int32)
    for ref, pbuf in ((eb0, posb0), (eb1, posb1)):
        for v in range(nvec):
            vec = ref[v]
            posv = jnp.zeros((LANES,), jnp.int32)
            for e in range(nexp):
                m = vec == e
                rank = plsc.cumsum(m.astype(jnp.int32)) - 1
                bc = jnp.sum(jnp.where(lane == e, base + counter, 0))
                posv = jnp.where(m, bc + rank, posv)
                counter = counter + jnp.where(
                    lane == e, plsc.all_reduce_population_count(m), 0)
            pbuf[v] = posv
    pltpu.sync_copy(posb0, pos0_hbm.at[pl.ds(rb, nvec)])
    pltpu.sync_copy(posb1, pos1_hbm.at[pl.ds(rb, nvec)])

    for c in range(nvec):
        tb = wid * tpw + c * LANES
        pltpu.sync_copy(xb_hbm.at[pl.ds(tb, LANES)], xbuf)
        cp0 = pltpu.async_copy(xbuf, xs_hbm.at[posb0.at[c]], sem)
        cp1 = pltpu.async_copy(xbuf, xs_hbm.at[posb1.at[c]], sem)
        cp0.wait()
        cp1.wait()


def _ffn_body(be_ref, na_ref, xs_ref, w1_ref, b1_ref, w2_ref, b2_ref,
              ys_ref):
    g = pl.program_id(0)

    @pl.when(g < na_ref[0])
    def _():
        h = jnp.dot(xs_ref[...].astype(jnp.bfloat16), w1_ref[0],
                    preferred_element_type=jnp.float32) + b1_ref[0]
        h = 0.5 * h * (1.0 + lax.erf(h * 0.7071067811865476))
        ys_ref[...] = jnp.dot(h.astype(jnp.bfloat16), w2_ref[0],
                              preferred_element_type=jnp.float32) \
            + b2_ref[0]


def _gather_body(ys_hbm, pos0_hbm, pos1_hbm, ys0_hbm, ys1_hbm, posb0,
                 posb1, ybuf0, ybuf1, sem, *, tpw):
    wid = lax.axis_index("s") * 2 + lax.axis_index("c")
    nvec = tpw // LANES
    rb = wid * nvec
    pltpu.sync_copy(pos0_hbm.at[pl.ds(rb, nvec)], posb0)
    pltpu.sync_copy(pos1_hbm.at[pl.ds(rb, nvec)], posb1)
    for c in range(nvec):
        tb = wid * tpw + c * LANES
        g0 = pltpu.async_copy(ys_hbm.at[posb0.at[c]], ybuf0, sem)
        g1 = pltpu.async_copy(ys_hbm.at[posb1.at[c]], ybuf1, sem)
        g0.wait()
        g1.wait()
        pltpu.sync_copy(ybuf0, ys0_hbm.at[pl.ds(tb, LANES)])
        pltpu.sync_copy(ybuf1, ys1_hbm.at[pl.ds(tb, LANES)])


def _out_body(x_ref, y0_ref, y1_ref, w0_ref, w1_ref, gamma_ref, beta_ref,
              o_ref):
    y = x_ref[...] + w0_ref[...] * y0_ref[...] + w1_ref[...] * y1_ref[...]
    mu = jnp.mean(y, axis=1, keepdims=True)
    var = jnp.mean((y - mu) ** 2, axis=1, keepdims=True)
    o_ref[...] = (y - mu) / jnp.sqrt(var + 1e-5) * gamma_ref[...] \
        + beta_ref[...]


def kernel(x, gate_w, gate_b, w1, b1, w2, b2, gamma, beta):
    B, T, D = x.shape
    E = gate_w.shape[1]
    H = w1.shape[2]
    gmax = (2 * T) // BLK + E - 1
    nrows = gmax * BLK
    tpw = T // NWORK

    x2 = x.reshape(T, D)
    w1b = w1.astype(jnp.bfloat16)
    w2b = w2.astype(jnp.bfloat16)
    b1r = b1.reshape(E, 1, H)
    b2r = b2.reshape(E, 1, D)
    gbr = gate_b.reshape(1, E)
    gammar = gamma.reshape(1, D)
    betar = beta.reshape(1, D)

    shift = (tpw - 1).bit_length()
    e0, e1, w0c, w1c, histtab = pl.pallas_call(
        functools.partial(_gate_body, shift=shift),
        in_specs=[pl.BlockSpec((T, D), lambda: (0, 0)),
                  pl.BlockSpec((D, E), lambda: (0, 0)),
                  pl.BlockSpec((1, E), lambda: (0, 0))],
        out_specs=[pl.BlockSpec((T, 1), lambda: (0, 0)),
                   pl.BlockSpec((T, 1), lambda: (0, 0)),
                   pl.BlockSpec((T, 1), lambda: (0, 0)),
                   pl.BlockSpec((T, 1), lambda: (0, 0)),
                   pl.BlockSpec((NWORK, LANES), lambda: (0, 0))],
        out_shape=[jax.ShapeDtypeStruct((T, 1), jnp.int32),
                   jax.ShapeDtypeStruct((T, 1), jnp.int32),
                   jax.ShapeDtypeStruct((T, 1), jnp.float32),
                   jax.ShapeDtypeStruct((T, 1), jnp.float32),
                   jax.ShapeDtypeStruct((NWORK, LANES), jnp.int32)],
    )(x2, gate_w, gbr)

    e0r = e0.reshape(T // LANES, LANES)
    e1r = e1.reshape(T // LANES, LANES)

    mesh = plsc.VectorSubcoreMesh(core_axis_name="c", subcore_axis_name="s",
                                  num_cores=2)
    route = pl.kernel(
        functools.partial(_route_body, tpw=tpw, nexp=E),
        mesh=mesh,
        compiler_params=pltpu.CompilerParams(needs_layout_passes=False),
        out_type=(jax.ShapeDtypeStruct((T // LANES, LANES), jnp.int32),
                  jax.ShapeDtypeStruct((T // LANES, LANES), jnp.int32),
                  jax.ShapeDtypeStruct((nrows, D), jnp.float32),
                  jax.ShapeDtypeStruct((2, LANES), jnp.int32),
                  jax.ShapeDtypeStruct((1, LANES), jnp.int32)),
        scratch_types=[
            pltpu.VMEM((tpw // LANES, LANES), jnp.int32),
            pltpu.VMEM((tpw // LANES, LANES), jnp.int32),
            pltpu.VMEM((NWORK, LANES), jnp.int32),
            pltpu.VMEM((tpw // LANES, LANES), jnp.int32),
            pltpu.VMEM((tpw // LANES, LANES), jnp.int32),
            pltpu.VMEM((2, LANES), jnp.int32),
            pltpu.VMEM((1, LANES), jnp.int32),
            pltpu.VMEM((LANES, D), jnp.float32),
            pltpu.SemaphoreType.DMA,
        ],
    )
    pos0, pos1, xs, be2, na2 = route(e0r, e1r, histtab, x2)

    ys = pl.pallas_call(
        _ffn_body,
        grid_spec=pltpu.PrefetchScalarGridSpec(
            num_scalar_prefetch=2,
            grid=(gmax,),
            in_specs=[
                pl.BlockSpec((BLK, D), lambda g, be, na: (g, 0)),
                pl.BlockSpec((1, D, H), lambda g, be, na: (be[g], 0, 0)),
                pl.BlockSpec((1, 1, H), lambda g, be, na: (be[g], 0, 0)),
                pl.BlockSpec((1, H, D), lambda g, be, na: (be[g], 0, 0)),
                pl.BlockSpec((1, 1, D), lambda g, be, na: (be[g], 0, 0)),
            ],
            out_specs=pl.BlockSpec((BLK, D), lambda g, be, na: (g, 0)),
        ),
        out_shape=jax.ShapeDtypeStruct((nrows, D), jnp.float32),
    )(be2.reshape(2 * LANES), na2.reshape(LANES), xs, w1b, b1r, w2b, b2r)

    gather = pl.kernel(
        functools.partial(_gather_body, tpw=tpw),
        mesh=plsc.VectorSubcoreMesh(core_axis_name="c",
                                    subcore_axis_name="s", num_cores=2),
        compiler_params=pltpu.CompilerParams(needs_layout_passes=False),
        out_type=(jax.ShapeDtypeStruct((T, D), jnp.float32),
                  jax.ShapeDtypeStruct((T, D), jnp.float32)),
        scratch_types=[
            pltpu.VMEM((tpw // LANES, LANES), jnp.int32),
            pltpu.VMEM((tpw // LANES, LANES), jnp.int32),
            pltpu.VMEM((LANES, D), jnp.float32),
            pltpu.VMEM((LANES, D), jnp.float32),
            pltpu.SemaphoreType.DMA,
        ],
    )
    ys0, ys1 = gather(ys, pos0, pos1)

    out = pl.pallas_call(
        _out_body,
        in_specs=[pl.BlockSpec((T, D), lambda: (0, 0)),
                  pl.BlockSpec((T, D), lambda: (0, 0)),
                  pl.BlockSpec((T, D), lambda: (0, 0)),
                  pl.BlockSpec((T, 1), lambda: (0, 0)),
                  pl.BlockSpec((T, 1), lambda: (0, 0)),
                  pl.BlockSpec((1, D), lambda: (0, 0)),
                  pl.BlockSpec((1, D), lambda: (0, 0))],
        out_specs=pl.BlockSpec((T, D), lambda: (0, 0)),
        out_shape=jax.ShapeDtypeStruct((T, D), jnp.float32),
    )(x2, ys0, ys1, w0c, w1c, gammar, betar)
    return out.reshape(B, T, D)


# speedup vs baseline: 1.6200x; 1.0011x over previous
import functools

import jax
import jax.numpy as jnp
from jax import lax
from jax.experimental import pallas as pl
from jax.experimental.pallas import tpu as pltpu
from jax.experimental.pallas import tpu_sc as plsc

BLK = 256
NWORK = 32
LANES = 16


def _gate_body(x_ref, gw_ref, gb_ref, e0_ref, e1_ref, w0_ref, w1_ref,
               hist_ref, *, shift):
    T = x_ref.shape[0]
    E = gw_ref.shape[1]
    xv = x_ref[...]
    logits = jnp.dot(xv, gw_ref[...],
                     preferred_element_type=jnp.float32) + gb_ref[...]
    ids = lax.broadcasted_iota(jnp.int32, (T, E), 1)
    m1 = jnp.max(logits, axis=1, keepdims=True)
    i1 = jnp.min(jnp.where(logits == m1, ids, E), axis=1, keepdims=True)
    masked = jnp.where(ids == i1, -jnp.inf, logits)
    m2 = jnp.max(masked, axis=1, keepdims=True)
    i2 = jnp.min(jnp.where(masked == m2, ids, E), axis=1, keepdims=True)
    p1 = 1.0 / (1.0 + jnp.exp(m2 - m1))
    e0_ref[...] = i1
    e1_ref[...] = i2
    w0_ref[...] = p1
    w1_ref[...] = 1.0 - p1
    ids16 = lax.broadcasted_iota(jnp.int32, (T, LANES), 1)
    onehot = (ids16 == i1).astype(jnp.float32) \
        + (ids16 == i2).astype(jnp.float32)
    wrow = lax.broadcasted_iota(jnp.int32, (NWORK, T), 0)
    tcol = lax.broadcasted_iota(jnp.int32, (NWORK, T), 1)
    sel = ((tcol >> shift) == wrow).astype(jnp.float32)
    hist_ref[...] = jnp.dot(sel, onehot,
                            preferred_element_type=jnp.float32
                            ).astype(jnp.int32)


def _route_body(e0_hbm, e1_hbm, hist_hbm, xb_hbm, pos0_hbm, pos1_hbm,
                xs_hbm, be_hbm, na_hbm, eb0, eb1, histall, posb0, posb1,
                bebuf, nabuf, xbuf, ldsem, sem, *, tpw, nexp):
    wid = lax.axis_index("s") * 2 + lax.axis_index("c")
    rb = wid * (tpw // LANES)
    nvec = tpw // LANES
    lane = lax.broadcasted_iota(jnp.int32, (LANES,), 0)

    pltpu.sync_copy(e0_hbm.at[pl.ds(rb, nvec)], eb0)
    pltpu.sync_copy(e1_hbm.at[pl.ds(rb, nvec)], eb1)
    pltpu.sync_copy(hist_hbm, histall)

    tot = jnp.zeros((LANES,), jnp.int32)
    pref = jnp.zeros((LANES,), jnp.int32)
    for w in range(NWORK):
        row = histall[w]
        tot = tot + row
        pref = pref + row * jnp.where(w < wid, 1, 0)

    padded = ((tot + (BLK - 1)) >> 8) << 8
    start = plsc.cumsum(padded) - padded
    base = start + pref
    nblk = padded >> 8
    gstart = plsc.cumsum(nblk) - nblk
    na_scal = jnp.sum(nblk)

    bg0 = jnp.zeros((LANES,), jnp.int32)
    bg1 = jnp.zeros((LANES,), jnp.int32)
    gv0 = lane
    gv1 = lane + LANES
    for e in range(nexp):
        ge = jnp.sum(jnp.where(lane == e, gstart, 0))
        bg0 = bg0 + (gv0 >= ge).astype(jnp.int32)
        bg1 = bg1 + (gv1 >= ge).astype(jnp.int32)

    @pl.when(wid == 0)
    def _write_meta():
        bebuf[0] = bg0 - 1
        bebuf[1] = bg1 - 1
        nabuf[0] = jnp.zeros((LANES,), jnp.int32) + na_scal
        pltpu.sync_copy(bebuf, be_hbm)
        pltpu.sync_copy(nabuf, na_hbm)

    counter = jnp.zeros((LANES,), jnp.int32)
    for ref, pbuf in ((eb0, posb0), (eb1, posb1)):
        for v in range(nvec):
            vec = ref[v]
            posv = jnp.zeros((LANES,), jnp.int32)
            for e in range(nexp):
                m = vec == e
                rank = plsc.cumsum(m.astype(jnp.int32)) - 1
                bc = jnp.sum(jnp.where(lane == e, base + counter, 0))
                posv = jnp.where(m, bc + rank, posv)
                counter = counter + jnp.where(
                    lane == e, plsc.all_reduce_population_count(m), 0)
            pbuf[v] = posv
    pltpu.sync_copy(posb0, pos0_hbm.at[pl.ds(rb, nvec)])
    pltpu.sync_copy(posb1, pos1_hbm.at[pl.ds(rb, nvec)])

    tb0 = wid * tpw
    ld = pltpu.async_copy(xb_hbm.at[pl.ds(tb0, LANES)], xbuf.at[0], ldsem)
    for c in range(nvec):
        ld.wait()
        if c + 1 < nvec:
            nxt = tb0 + (c + 1) * LANES
            ld = pltpu.async_copy(xb_hbm.at[pl.ds(nxt, LANES)],
                                  xbuf.at[(c + 1) % 2], ldsem)
        cp0 = pltpu.async_copy(xbuf.at[c % 2], xs_hbm.at[posb0.at[c]], sem)
        cp1 = pltpu.async_copy(xbuf.at[c % 2], xs_hbm.at[posb1.at[c]], sem)
        cp0.wait()
        cp1.wait()


def _ffn_body(be_ref, na_ref, xs_ref, w1_ref, b1_ref, w2_ref, b2_ref,
              ys_ref):
    g = pl.program_id(0)

    @pl.when(g < na_ref[0])
    def _():
        h = jnp.dot(xs_ref[...].astype(jnp.bfloat16), w1_ref[0],
                    preferred_element_type=jnp.float32) + b1_ref[0]
        h = 0.5 * h * (1.0 + lax.erf(h * 0.7071067811865476))
        ys_ref[...] = jnp.dot(h.astype(jnp.bfloat16), w2_ref[0],
                              preferred_element_type=jnp.float32) \
            + b2_ref[0]


def _gather_body(ys_hbm, pos0_hbm, pos1_hbm, ys0_hbm, ys1_hbm, posb0,
                 posb1, ybuf0, ybuf1, sem, *, tpw):
    wid = lax.axis_index("s") * 2 + lax.axis_index("c")
    nvec = tpw // LANES
    rb = wid * nvec
    pltpu.sync_copy(pos0_hbm.at[pl.ds(rb, nvec)], posb0)
    pltpu.sync_copy(pos1_hbm.at[pl.ds(rb, nvec)], posb1)
    g0 = pltpu.async_copy(ys_hbm.at[posb0.at[0]], ybuf0.at[0], sem)
    g1 = pltpu.async_copy(ys_hbm.at[posb1.at[0]], ybuf1.at[0], sem)
    for c in range(nvec):
        g0.wait()
        g1.wait()
        if c + 1 < nvec:
            g0 = pltpu.async_copy(ys_hbm.at[posb0.at[c + 1]],
                                  ybuf0.at[(c + 1) % 2], sem)
            g1 = pltpu.async_copy(ys_hbm.at[posb1.at[c + 1]],
                                  ybuf1.at[(c + 1) % 2], sem)
        tb = wid * tpw + c * LANES
        pltpu.sync_copy(ybuf0.at[c % 2], ys0_hbm.at[pl.ds(tb, LANES)])
        pltpu.sync_copy(ybuf1.at[c % 2], ys1_hbm.at[pl.ds(tb, LANES)])


def _out_body(x_ref, y0_ref, y1_ref, w0_ref, w1_ref, gamma_ref, beta_ref,
              o_ref):
    y = x_ref[...] + w0_ref[...] * y0_ref[...] + w1_ref[...] * y1_ref[...]
    mu = jnp.mean(y, axis=1, keepdims=True)
    var = jnp.mean((y - mu) ** 2, axis=1, keepdims=True)
    o_ref[...] = (y - mu) / jnp.sqrt(var + 1e-5) * gamma_ref[...] \
        + beta_ref[...]


def kernel(x, gate_w, gate_b, w1, b1, w2, b2, gamma, beta):
    B, T, D = x.shape
    E = gate_w.shape[1]
    H = w1.shape[2]
    gmax = (2 * T) // BLK + E - 1
    nrows = gmax * BLK
    tpw = T // NWORK

    x2 = x.reshape(T, D)
    w1b = w1.astype(jnp.bfloat16)
    w2b = w2.astype(jnp.bfloat16)
    b1r = b1.reshape(E, 1, H)
    b2r = b2.reshape(E, 1, D)
    gbr = gate_b.reshape(1, E)
    gammar = gamma.reshape(1, D)
    betar = beta.reshape(1, D)

    shift = (tpw - 1).bit_length()
    e0, e1, w0c, w1c, histtab = pl.pallas_call(
        functools.partial(_gate_body, shift=shift),
        in_specs=[pl.BlockSpec((T, D), lambda: (0, 0)),
                  pl.BlockSpec((D, E), lambda: (0, 0)),
                  pl.BlockSpec((1, E), lambda: (0, 0))],
        out_specs=[pl.BlockSpec((T, 1), lambda: (0, 0)),
                   pl.BlockSpec((T, 1), lambda: (0, 0)),
                   pl.BlockSpec((T, 1), lambda: (0, 0)),
                   pl.BlockSpec((T, 1), lambda: (0, 0)),
                   pl.BlockSpec((NWORK, LANES), lambda: (0, 0))],
        out_shape=[jax.ShapeDtypeStruct((T, 1), jnp.int32),
                   jax.ShapeDtypeStruct((T, 1), jnp.int32),
                   jax.ShapeDtypeStruct((T, 1), jnp.float32),
                   jax.ShapeDtypeStruct((T, 1), jnp.float32),
                   jax.ShapeDtypeStruct((NWORK, LANES), jnp.int32)],
    )(x2, gate_w, gbr)

    e0r = e0.reshape(T // LANES, LANES)
    e1r = e1.reshape(T // LANES, LANES)

    mesh = plsc.VectorSubcoreMesh(core_axis_name="c", subcore_axis_name="s",
                                  num_cores=2)
    route = pl.kernel(
        functools.partial(_route_body, tpw=tpw, nexp=E),
        mesh=mesh,
        compiler_params=pltpu.CompilerParams(needs_layout_passes=False),
        out_type=(jax.ShapeDtypeStruct((T // LANES, LANES), jnp.int32),
                  jax.ShapeDtypeStruct((T // LANES, LANES), jnp.int32),
                  jax.ShapeDtypeStruct((nrows, D), jnp.float32),
                  jax.ShapeDtypeStruct((2, LANES), jnp.int32),
                  jax.ShapeDtypeStruct((1, LANES), jnp.int32)),
        scratch_types=[
            pltpu.VMEM((tpw // LANES, LANES), jnp.int32),
            pltpu.VMEM((tpw // LANES, LANES), jnp.int32),
            pltpu.VMEM((NWORK, LANES), jnp.int32),
            pltpu.VMEM((tpw // LANES, LANES), jnp.int32),
            pltpu.VMEM((tpw // LANES, LANES), jnp.int32),
            pltpu.VMEM((2, LANES), jnp.int32),
            pltpu.VMEM((1, LANES), jnp.int32),
            pltpu.VMEM((2, LANES, D), jnp.float32),
            pltpu.SemaphoreType.DMA,
            pltpu.SemaphoreType.DMA,
        ],
    )
    pos0, pos1, xs, be2, na2 = route(e0r, e1r, histtab, x2)

    ys = pl.pallas_call(
        _ffn_body,
        grid_spec=pltpu.PrefetchScalarGridSpec(
            num_scalar_prefetch=2,
            grid=(gmax,),
            in_specs=[
                pl.BlockSpec((BLK, D), lambda g, be, na: (g, 0)),
                pl.BlockSpec((1, D, H), lambda g, be, na: (be[g], 0, 0)),
                pl.BlockSpec((1, 1, H), lambda g, be, na: (be[g], 0, 0)),
                pl.BlockSpec((1, H, D), lambda g, be, na: (be[g], 0, 0)),
                pl.BlockSpec((1, 1, D), lambda g, be, na: (be[g], 0, 0)),
            ],
            out_specs=pl.BlockSpec((BLK, D), lambda g, be, na: (g, 0)),
        ),
        out_shape=jax.ShapeDtypeStruct((nrows, D), jnp.float32),
    )(be2.reshape(2 * LANES), na2.reshape(LANES), xs, w1b, b1r, w2b, b2r)

    gather = pl.kernel(
        functools.partial(_gather_body, tpw=tpw),
        mesh=plsc.VectorSubcoreMesh(core_axis_name="c",
                                    subcore_axis_name="s", num_cores=2),
        compiler_params=pltpu.CompilerParams(needs_layout_passes=False),
        out_type=(jax.ShapeDtypeStruct((T, D), jnp.float32),
                  jax.ShapeDtypeStruct((T, D), jnp.float32)),
        scratch_types=[
            pltpu.VMEM((tpw // LANES, LANES), jnp.int32),
            pltpu.VMEM((tpw // LANES, LANES), jnp.int32),
            pltpu.VMEM((2, LANES, D), jnp.float32),
            pltpu.VMEM((2, LANES, D), jnp.float32),
            pltpu.SemaphoreType.DMA,
        ],
    )
    ys0, ys1 = gather(ys, pos0, pos1)

    out = pl.pallas_call(
        _out_body,
        in_specs=[pl.BlockSpec((T, D), lambda: (0, 0)),
                  pl.BlockSpec((T, D), lambda: (0, 0)),
                  pl.BlockSpec((T, D), lambda: (0, 0)),
                  pl.BlockSpec((T, 1), lambda: (0, 0)),
                  pl.BlockSpec((T, 1), lambda: (0, 0)),
                  pl.BlockSpec((1, D), lambda: (0, 0)),
                  pl.BlockSpec((1, D), lambda: (0, 0))],
        out_specs=pl.BlockSpec((T, D), lambda: (0, 0)),
        out_shape=jax.ShapeDtypeStruct((T, D), jnp.float32),
    )(x2, ys0, ys1, w0c, w1c, gammar, betar)
    return out.reshape(B, T, D)


# speedup vs baseline: 1.6298x; 1.0060x over previous
import functools

import jax
import jax.numpy as jnp
from jax import lax
from jax.experimental import pallas as pl
from jax.experimental.pallas import tpu as pltpu
from jax.experimental.pallas import tpu_sc as plsc

BLK = 256
NWORK = 32
LANES = 16


def _gate_body(x_ref, gw_ref, gb_ref, e0_ref, e1_ref, w0_ref, w1_ref,
               hist_ref, *, shift):
    T = x_ref.shape[0]
    E = gw_ref.shape[1]
    xv = x_ref[...]
    logits = jnp.dot(xv, gw_ref[...],
                     preferred_element_type=jnp.float32) + gb_ref[...]
    ids = lax.broadcasted_iota(jnp.int32, (T, E), 1)
    m1 = jnp.max(logits, axis=1, keepdims=True)
    i1 = jnp.min(jnp.where(logits == m1, ids, E), axis=1, keepdims=True)
    masked = jnp.where(ids == i1, -jnp.inf, logits)
    m2 = jnp.max(masked, axis=1, keepdims=True)
    i2 = jnp.min(jnp.where(masked == m2, ids, E), axis=1, keepdims=True)
    p1 = 1.0 / (1.0 + jnp.exp(m2 - m1))
    e0_ref[...] = i1
    e1_ref[...] = i2
    w0_ref[...] = p1
    w1_ref[...] = 1.0 - p1
    ids16 = lax.broadcasted_iota(jnp.int32, (T, LANES), 1)
    onehot = (ids16 == i1).astype(jnp.float32) \
        + (ids16 == i2).astype(jnp.float32)
    wrow = lax.broadcasted_iota(jnp.int32, (NWORK, T), 0)
    tcol = lax.broadcasted_iota(jnp.int32, (NWORK, T), 1)
    sel = ((tcol >> shift) == wrow).astype(jnp.float32)
    hist_ref[...] = jnp.dot(sel, onehot,
                            preferred_element_type=jnp.float32
                            ).astype(jnp.int32)


def _route_body(e0_hbm, e1_hbm, hist_hbm, xb_hbm, pos0_hbm, pos1_hbm,
                xs_hbm, be_hbm, na_hbm, eb0, eb1, histall, posb0, posb1,
                bebuf, nabuf, xbuf, ldsem, sem, *, tpw, nexp):
    wid = lax.axis_index("s") * 2 + lax.axis_index("c")
    rb = wid * (tpw // LANES)
    nvec = tpw // LANES
    lane = lax.broadcasted_iota(jnp.int32, (LANES,), 0)

    pltpu.sync_copy(e0_hbm.at[pl.ds(rb, nvec)], eb0)
    pltpu.sync_copy(e1_hbm.at[pl.ds(rb, nvec)], eb1)
    pltpu.sync_copy(hist_hbm, histall)

    tot = jnp.zeros((LANES,), jnp.int32)
    pref = jnp.zeros((LANES,), jnp.int32)
    for w in range(NWORK):
        row = histall[w]
        tot = tot + row
        pref = pref + row * jnp.where(w < wid, 1, 0)

    padded = ((tot + (BLK - 1)) >> 8) << 8
    start = plsc.cumsum(padded) - padded
    base = start + pref
    nblk = padded >> 8
    gstart = plsc.cumsum(nblk) - nblk
    na_scal = jnp.sum(nblk)

    bg0 = jnp.zeros((LANES,), jnp.int32)
    bg1 = jnp.zeros((LANES,), jnp.int32)
    gv0 = lane
    gv1 = lane + LANES
    for e in range(nexp):
        ge = jnp.sum(jnp.where(lane == e, gstart, 0))
        bg0 = bg0 + (gv0 >= ge).astype(jnp.int32)
        bg1 = bg1 + (gv1 >= ge).astype(jnp.int32)

    @pl.when(wid == 0)
    def _write_meta():
        bebuf[0] = bg0 - 1
        bebuf[1] = bg1 - 1
        nabuf[0] = jnp.zeros((LANES,), jnp.int32) + na_scal
        pltpu.sync_copy(bebuf, be_hbm)
        pltpu.sync_copy(nabuf, na_hbm)

    counter = jnp.zeros((LANES,), jnp.int32)
    for ref, pbuf in ((eb0, posb0), (eb1, posb1)):
        for v in range(nvec):
            vec = ref[v]
            posv = jnp.zeros((LANES,), jnp.int32)
            for e in range(nexp):
                m = vec == e
                rank = plsc.cumsum(m.astype(jnp.int32)) - 1
                bc = jnp.sum(jnp.where(lane == e, base + counter, 0))
                posv = jnp.where(m, bc + rank, posv)
                counter = counter + jnp.where(
                    lane == e, plsc.all_reduce_population_count(m), 0)
            pbuf[v] = posv
    pltpu.sync_copy(posb0, pos0_hbm.at[pl.ds(rb, nvec)])
    pltpu.sync_copy(posb1, pos1_hbm.at[pl.ds(rb, nvec)])

    tb0 = wid * tpw
    ld = pltpu.async_copy(xb_hbm.at[pl.ds(tb0, LANES)], xbuf.at[0], ldsem)
    for c in range(nvec):
        ld.wait()
        if c + 1 < nvec:
            nxt = tb0 + (c + 1) * LANES
            ld = pltpu.async_copy(xb_hbm.at[pl.ds(nxt, LANES)],
                                  xbuf.at[(c + 1) % 2], ldsem)
        cp0 = pltpu.async_copy(xbuf.at[c % 2], xs_hbm.at[posb0.at[c]], sem)
        cp1 = pltpu.async_copy(xbuf.at[c % 2], xs_hbm.at[posb1.at[c]], sem)
        cp0.wait()
        cp1.wait()


def _ffn_body(be_ref, na_ref, xs_ref, w1_ref, b1_ref, w2_ref, b2_ref,
              ys_ref):
    g = pl.program_id(0)

    @pl.when(g < na_ref[0])
    def _():
        h = jnp.dot(xs_ref[...].astype(jnp.bfloat16), w1_ref[0],
                    preferred_element_type=jnp.float32) + b1_ref[0]
        h = 0.5 * h * (1.0 + lax.erf(h * 0.7071067811865476))
        ys_ref[...] = jnp.dot(h.astype(jnp.bfloat16), w2_ref[0],
                              preferred_element_type=jnp.float32) \
            + b2_ref[0]


def _gather_body(ys_hbm, pos0_hbm, pos1_hbm, ys0_hbm, ys1_hbm, posb0,
                 posb1, ybuf0, ybuf1, sem, *, tpw):
    wid = lax.axis_index("s") * 2 + lax.axis_index("c")
    nvec = tpw // LANES
    rb = wid * nvec
    pltpu.sync_copy(pos0_hbm.at[pl.ds(rb, nvec)], posb0)
    pltpu.sync_copy(pos1_hbm.at[pl.ds(rb, nvec)], posb1)
    g0 = pltpu.async_copy(ys_hbm.at[posb0.at[0]], ybuf0.at[0], sem)
    g1 = pltpu.async_copy(ys_hbm.at[posb1.at[0]], ybuf1.at[0], sem)
    for c in range(nvec):
        g0.wait()
        g1.wait()
        if c + 1 < nvec:
            g0 = pltpu.async_copy(ys_hbm.at[posb0.at[c + 1]],
                                  ybuf0.at[(c + 1) % 2], sem)
            g1 = pltpu.async_copy(ys_hbm.at[posb1.at[c + 1]],
                                  ybuf1.at[(c + 1) % 2], sem)
        tb = wid * tpw + c * LANES
        pltpu.sync_copy(ybuf0.at[c % 2], ys0_hbm.at[pl.ds(tb, LANES)])
        pltpu.sync_copy(ybuf1.at[c % 2], ys1_hbm.at[pl.ds(tb, LANES)])


def _out_body(x_ref, y0_ref, y1_ref, w0_ref, w1_ref, gamma_ref, beta_ref,
              o_ref):
    y = x_ref[...] + w0_ref[...] * y0_ref[...] + w1_ref[...] * y1_ref[...]
    mu = jnp.mean(y, axis=1, keepdims=True)
    var = jnp.mean((y - mu) ** 2, axis=1, keepdims=True)
    o_ref[...] = (y - mu) / jnp.sqrt(var + 1e-5) * gamma_ref[...] \
        + beta_ref[...]


def kernel(x, gate_w, gate_b, w1, b1, w2, b2, gamma, beta):
    B, T, D = x.shape
    E = gate_w.shape[1]
    H = w1.shape[2]
    gmax = (2 * T) // BLK + E - 1
    nrows = gmax * BLK
    tpw = T // NWORK

    x2 = x.reshape(T, D)
    w1b = w1.astype(jnp.bfloat16)
    w2b = w2.astype(jnp.bfloat16)
    b1r = b1.reshape(E, 1, H)
    b2r = b2.reshape(E, 1, D)
    gbr = gate_b.reshape(1, E)
    gammar = gamma.reshape(1, D)
    betar = beta.reshape(1, D)

    shift = (tpw - 1).bit_length()
    e0, e1, w0c, w1c, histtab = pl.pallas_call(
        functools.partial(_gate_body, shift=shift),
        in_specs=[pl.BlockSpec((T, D), lambda: (0, 0)),
                  pl.BlockSpec((D, E), lambda: (0, 0)),
                  pl.BlockSpec((1, E), lambda: (0, 0))],
        out_specs=[pl.BlockSpec((T, 1), lambda: (0, 0)),
                   pl.BlockSpec((T, 1), lambda: (0, 0)),
                   pl.BlockSpec((T, 1), lambda: (0, 0)),
                   pl.BlockSpec((T, 1), lambda: (0, 0)),
                   pl.BlockSpec((NWORK, LANES), lambda: (0, 0))],
        out_shape=[jax.ShapeDtypeStruct((T, 1), jnp.int32),
                   jax.ShapeDtypeStruct((T, 1), jnp.int32),
                   jax.ShapeDtypeStruct((T, 1), jnp.float32),
                   jax.ShapeDtypeStruct((T, 1), jnp.float32),
                   jax.ShapeDtypeStruct((NWORK, LANES), jnp.int32)],
    )(x2, gate_w, gbr)

    e0r = e0.reshape(T // LANES, LANES)
    e1r = e1.reshape(T // LANES, LANES)

    mesh = plsc.VectorSubcoreMesh(core_axis_name="c", subcore_axis_name="s",
                                  num_cores=2)
    route = pl.kernel(
        functools.partial(_route_body, tpw=tpw, nexp=E),
        mesh=mesh,
        compiler_params=pltpu.CompilerParams(needs_layout_passes=False),
        out_type=(jax.ShapeDtypeStruct((T // LANES, LANES), jnp.int32),
                  jax.ShapeDtypeStruct((T // LANES, LANES), jnp.int32),
                  jax.ShapeDtypeStruct((nrows, D), jnp.float32),
                  jax.ShapeDtypeStruct((2, LANES), jnp.int32),
                  jax.ShapeDtypeStruct((1, LANES), jnp.int32)),
        scratch_types=[
            pltpu.VMEM((tpw // LANES, LANES), jnp.int32),
            pltpu.VMEM((tpw // LANES, LANES), jnp.int32),
            pltpu.VMEM((NWORK, LANES), jnp.int32),
            pltpu.VMEM((tpw // LANES, LANES), jnp.int32),
            pltpu.VMEM((tpw // LANES, LANES), jnp.int32),
            pltpu.VMEM((2, LANES), jnp.int32),
            pltpu.VMEM((1, LANES), jnp.int32),
            pltpu.VMEM((2, LANES, D), jnp.float32),
            pltpu.SemaphoreType.DMA,
            pltpu.SemaphoreType.DMA,
        ],
    )
    pos0, pos1, xs, be2, na2 = route(e0r, e1r, histtab, x2)

    ys = pl.pallas_call(
        _ffn_body,
        grid_spec=pltpu.PrefetchScalarGridSpec(
            num_scalar_prefetch=2,
            grid=(gmax,),
            in_specs=[
                pl.BlockSpec((BLK, D), lambda g, be, na: (g, 0)),
                pl.BlockSpec((1, D, H), lambda g, be, na: (be[g], 0, 0)),
                pl.BlockSpec((1, 1, H), lambda g, be, na: (be[g], 0, 0)),
                pl.BlockSpec((1, H, D), lambda g, be, na: (be[g], 0, 0)),
                pl.BlockSpec((1, 1, D), lambda g, be, na: (be[g], 0, 0)),
            ],
            out_specs=pl.BlockSpec((BLK, D), lambda g, be, na: (g, 0)),
        ),
        out_shape=jax.ShapeDtypeStruct((nrows, D), jnp.float32),
    )(be2.reshape(2 * LANES), na2.reshape(LANES), xs, w1b, b1r, w2b, b2r)

    gather = pl.kernel(
        functools.partial(_gather_body, tpw=tpw),
        mesh=plsc.VectorSubcoreMesh(core_axis_name="c",
                                    subcore_axis_name="s", num_cores=2),
        compiler_params=pltpu.CompilerParams(needs_layout_passes=False),
        out_type=(jax.ShapeDtypeStruct((T, D), jnp.float32),
                  jax.ShapeDtypeStruct((T, D), jnp.float32)),
        scratch_types=[
            pltpu.VMEM((tpw // LANES, LANES), jnp.int32),
            pltpu.VMEM((tpw // LANES, LANES), jnp.int32),
            pltpu.VMEM((2, LANES, D), jnp.float32),
            pltpu.VMEM((2, LANES, D), jnp.float32),
            pltpu.SemaphoreType.DMA,
        ],
    )
    ys0, ys1 = gather(ys, pos0, pos1)

    TB = 256
    out = pl.pallas_call(
        _out_body,
        grid=(T // TB,),
        in_specs=[pl.BlockSpec((TB, D), lambda i: (i, 0)),
                  pl.BlockSpec((TB, D), lambda i: (i, 0)),
                  pl.BlockSpec((TB, D), lambda i: (i, 0)),
                  pl.BlockSpec((TB, 1), lambda i: (i, 0)),
                  pl.BlockSpec((TB, 1), lambda i: (i, 0)),
                  pl.BlockSpec((1, D), lambda i: (0, 0)),
                  pl.BlockSpec((1, D), lambda i: (0, 0))],
        out_specs=pl.BlockSpec((TB, D), lambda i: (i, 0)),
        out_shape=jax.ShapeDtypeStruct((T, D), jnp.float32),
    )(x2, ys0, ys1, w0c, w1c, gammar, betar)
    return out.reshape(B, T, D)
